# Initial kernel scaffold; baseline (speedup 1.0000x reference)
import jax
import jax.numpy as jnp
from jax.experimental import pallas as pl


def kernel(hidden_states, w_router1, w_gate_up1, w_down1, w_router2, w_gate_up2, w_down2, w_router3, w_gate_up3, w_down3, w_router4, w_gate_up4, w_down4):
    raise NotImplementedError("write your pallas kernel here")


# speedup vs baseline: 1.2766x; 1.2766x over previous
import functools

import jax
import jax.numpy as jnp
from jax import lax
from jax.experimental import pallas as pl
from jax.experimental.pallas import tpu as pltpu
from jax.experimental.pallas import tpu_sc as plsc

E = 8
TOP_K = 2
D = 1024
F = 1024
CAP = 1024
T = 2048
BLK = 256
NBLK = 24
BUF = NBLK * BLK
BUFP = BUF + 64
TRASH = BUF + 32

_f32 = jnp.float32
_i32 = jnp.int32


def _router_body(x, wr, outs):
    (d1s_ref, d2s_ref, d1g_ref, d2g_ref, w1_ref, w2_ref, meta_ref, x_ref_out) = outs
    logits = lax.dot_general(x, wr[...], (((1,), (1,)), ((), ())),
                             preferred_element_type=_f32)
    iota_e = lax.broadcasted_iota(_i32, (T, E), 1)
    l1 = jnp.max(logits, axis=1, keepdims=True)
    i1 = jnp.min(jnp.where(logits >= l1, iota_e, E), axis=1, keepdims=True)
    masked = jnp.where(iota_e == i1, -jnp.inf, logits)
    l2 = jnp.max(masked, axis=1, keepdims=True)
    i2 = jnp.min(jnp.where(masked >= l2, iota_e, E), axis=1, keepdims=True)
    s = jnp.exp(l2 - l1)
    w1 = 1.0 / (1.0 + s)
    w2 = s / (1.0 + s)

    hot1 = (iota_e == i1)
    hot2 = (iota_e == i2)
    H = hot1.astype(_f32) + hot2.astype(_f32)

    r = lax.broadcasted_iota(_i32, (128, 128), 0)
    c = lax.broadcasted_iota(_i32, (128, 128), 1)
    Lstrict = (c < r).astype(_f32)
    tot = jnp.zeros((1, E), _f32)
    blocks = []
    for b in range(T // 128):
        Hb = H[b * 128:(b + 1) * 128, :]
        Cb = jnp.dot(Lstrict, Hb, preferred_element_type=_f32) + tot
        blocks.append(Cb)
        tot = tot + jnp.sum(Hb, axis=0, keepdims=True)
    C = jnp.concatenate(blocks, axis=0)

    counts = tot
    kept = jnp.minimum(counts, float(CAP))
    padded = jnp.floor((kept + float(BLK - 1)) / float(BLK)) * float(BLK)
    e_r = lax.broadcasted_iota(_i32, (E, E), 0)
    e_c = lax.broadcasted_iota(_i32, (E, E), 1)
    U = (e_r < e_c).astype(_f32)
    po = jnp.dot(padded, U, preferred_element_type=_f32)
    padded_total = jnp.sum(padded, axis=1, keepdims=True)

    pos1 = jnp.sum(jnp.where(hot1, C, 0.0), axis=1, keepdims=True)
    pos2 = jnp.sum(jnp.where(hot2, C, 0.0), axis=1, keepdims=True)
    off1 = jnp.sum(jnp.where(hot1, jnp.broadcast_to(po, (T, E)), 0.0),
                   axis=1, keepdims=True)
    off2 = jnp.sum(jnp.where(hot2, jnp.broadcast_to(po, (T, E)), 0.0),
                   axis=1, keepdims=True)
    v1 = pos1 < float(CAP)
    v2 = pos2 < float(CAP)
    dest1 = (off1 + pos1).astype(_i32)
    dest2 = (off2 + pos2).astype(_i32)
    d1s_ref[...] = jnp.where(v1, dest1, TRASH)
    d2s_ref[...] = jnp.where(v2, dest2, TRASH)
    d1g_ref[...] = jnp.where(v1, dest1, 0)
    d2g_ref[...] = jnp.where(v2, dest2, 0)
    w1_ref[...] = jnp.where(v1, w1, 0.0)
    w2_ref[...] = jnp.where(v2, w2, 0.0)

    poT = lax.dot_general(jnp.eye(E, dtype=_f32), po, (((1,), (1,)), ((), ())),
                          preferred_element_type=_f32)
    bs = lax.broadcasted_iota(_i32, (1, 32), 1).astype(_f32) * float(BLK)
    eb = jnp.sum((jnp.broadcast_to(poT, (E, 32)) <=
                  jnp.broadcast_to(bs, (E, 32))).astype(_f32),
                 axis=0, keepdims=True) - 1.0
    eb = jnp.clip(eb, 0.0, float(E - 1))
    n_act = padded_total / float(BLK)
    meta = jnp.concatenate([eb, jnp.broadcast_to(n_act, (1, 32))], axis=1)
    meta_ref[...] = meta.astype(_i32)
    if x_ref_out is not None:
        x_ref_out[...] = x


def _router_first_kernel(x_ref, wr_ref, d1s, d2s, d1g, d2g, w1, w2, meta):
    _router_body(x_ref[...], wr_ref, (d1s, d2s, d1g, d2g, w1, w2, meta, None))


def _router_combine_kernel(y1_ref, y2_ref, w1p_ref, w2p_ref, wr_ref,
                           d1s, d2s, d1g, d2g, w1, w2, meta, x_out):
    x = y1_ref[...] * w1p_ref[...] + y2_ref[...] * w2p_ref[...]
    _router_body(x, wr_ref, (d1s, d2s, d1g, d2g, w1, w2, meta, x_out))


_router_outs = [
    jax.ShapeDtypeStruct((T, 1), _i32),
    jax.ShapeDtypeStruct((T, 1), _i32),
    jax.ShapeDtypeStruct((T, 1), _i32),
    jax.ShapeDtypeStruct((T, 1), _i32),
    jax.ShapeDtypeStruct((T, 1), _f32),
    jax.ShapeDtypeStruct((T, 1), _f32),
    jax.ShapeDtypeStruct((1, 64), _i32),
]


def _route_first(x, wr):
    return pl.pallas_call(
        _router_first_kernel,
        out_shape=_router_outs,
    )(x, wr)


def _route_combine(y1, y2, w1p, w2p, wr):
    return pl.pallas_call(
        _router_combine_kernel,
        out_shape=_router_outs + [jax.ShapeDtypeStruct((T, D), _f32)],
    )(y1, y2, w1p, w2p, wr)


_PER_W = BUF // 32
_GCH = 64


def _dispatch_kernel(d1s_hbm, d2s_hbm, x_hbm, buf_hbm,
                     d1v, d2v, slot_v, idx_v, rows_v, shared_slot, sem):
    cid = lax.axis_index("c")
    sid = lax.axis_index("s")

    @pl.when(sid == 0)
    def _build():
        pltpu.sync_copy(d1s_hbm, d1v)
        pltpu.sync_copy(d2s_hbm, d2v)

        def _zero(i, _):
            slot_v[pl.ds(i * 16, 16)] = jnp.zeros((16,), _i32)
            return 0
        lax.fori_loop(0, BUFP // 16, _zero, 0)

        def _scat1(i, _):
            base = i * 16
            toks = lax.iota(_i32, 16) + base
            plsc.store_scatter(slot_v, [d1v[pl.ds(base, 16)]], toks)
            plsc.store_scatter(slot_v, [d2v[pl.ds(base, 16)]], toks)
            return 0
        lax.fori_loop(0, T // 16, _scat1, 0)
        pltpu.sync_copy(slot_v, shared_slot)

    plsc.subcore_barrier()

    wid = sid * 2 + cid
    for ch in range(_PER_W // _GCH):
        start = wid * _PER_W + ch * _GCH
        pltpu.sync_copy(shared_slot.at[pl.ds(start, _GCH)], idx_v)
        pltpu.async_copy(x_hbm.at[idx_v], rows_v, sem).wait()
        pltpu.sync_copy(rows_v, buf_hbm.at[pl.ds(start, _GCH)])


@functools.cache
def _get_dispatch():
    mesh = plsc.VectorSubcoreMesh(core_axis_name="c", subcore_axis_name="s")
    return pl.kernel(
        _dispatch_kernel,
        out_type=jax.ShapeDtypeStruct((BUF, D), _f32),
        mesh=mesh,
        scratch_types=[
            pltpu.VMEM((T,), _i32),
            pltpu.VMEM((T,), _i32),
            pltpu.VMEM((BUFP,), _i32),
            pltpu.VMEM((_GCH,), _i32),
            pltpu.VMEM((_GCH, D), _f32),
            pltpu.VMEM_SHARED((BUFP,), _i32),
            pltpu.SemaphoreType.DMA,
        ],
        compiler_params=pltpu.CompilerParams(needs_layout_passes=False),
    )


def _ffn_kernel(meta_ref, buf_ref, wgu_ref, wd_ref, y_ref):
    b = pl.program_id(0)

    @pl.when(b < meta_ref[32])
    def _():
        xb = buf_ref[...]
        gu = jnp.dot(xb, wgu_ref[0], preferred_element_type=_f32)
        g = gu[:, :F]
        u = gu[:, F:]
        h = g / (1.0 + jnp.exp(-g)) * u
        y_ref[...] = jnp.dot(h, wd_ref[0], preferred_element_type=_f32)


def _ffn(meta, buf, wgu, wd):
    grid_spec = pltpu.PrefetchScalarGridSpec(
        num_scalar_prefetch=1,
        grid=(NBLK,),
        in_specs=[
            pl.BlockSpec((BLK, D), lambda b, m: (b, 0)),
            pl.BlockSpec((1, D, 2 * F), lambda b, m: (m[b], 0, 0)),
            pl.BlockSpec((1, F, D), lambda b, m: (m[b], 0, 0)),
        ],
        out_specs=pl.BlockSpec((BLK, D), lambda b, m: (b, 0)),
    )
    return pl.pallas_call(
        _ffn_kernel,
        grid_spec=grid_spec,
        out_shape=jax.ShapeDtypeStruct((BUF, D), _f32),
    )(meta, buf, wgu, wd)


_TPW = T // 32


def _combine_kernel(y_hbm, d1g_hbm, d2g_hbm, y1_hbm, y2_hbm,
                    idx_v, rows_v, sem):
    cid = lax.axis_index("c")
    sid = lax.axis_index("s")
    wid = sid * 2 + cid
    base = wid * _TPW
    pltpu.sync_copy(d1g_hbm.at[pl.ds(base, _TPW)], idx_v)
    pltpu.async_copy(y_hbm.at[idx_v], rows_v, sem).wait()
    pltpu.sync_copy(rows_v, y1_hbm.at[pl.ds(base, _TPW)])
    pltpu.sync_copy(d2g_hbm.at[pl.ds(base, _TPW)], idx_v)
    pltpu.async_copy(y_hbm.at[idx_v], rows_v, sem).wait()
    pltpu.sync_copy(rows_v, y2_hbm.at[pl.ds(base, _TPW)])


@functools.cache
def _get_combine():
    mesh = plsc.VectorSubcoreMesh(core_axis_name="c", subcore_axis_name="s")
    return pl.kernel(
        _combine_kernel,
        out_type=[jax.ShapeDtypeStruct((T, D), _f32),
                  jax.ShapeDtypeStruct((T, D), _f32)],
        mesh=mesh,
        scratch_types=[
            pltpu.VMEM((_TPW,), _i32),
            pltpu.VMEM((_TPW, D), _f32),
            pltpu.SemaphoreType.DMA,
        ],
    )


def _final_sum_kernel(y1_ref, y2_ref, w1p_ref, w2p_ref, o_ref):
    o_ref[...] = y1_ref[...] * w1p_ref[...] + y2_ref[...] * w2p_ref[...]


def _final_sum(y1, y2, w1p, w2p):
    return pl.pallas_call(
        _final_sum_kernel,
        out_shape=jax.ShapeDtypeStruct((T, D), _f32),
    )(y1, y2, w1p, w2p)


def kernel(hidden_states, w_router1, w_gate_up1, w_down1,
           w_router2, w_gate_up2, w_down2,
           w_router3, w_gate_up3, w_down3,
           w_router4, w_gate_up4, w_down4):
    B, S, Dm = hidden_states.shape
    x = hidden_states.reshape(B * S, Dm)
    layers = [(w_router1, w_gate_up1, w_down1),
              (w_router2, w_gate_up2, w_down2),
              (w_router3, w_gate_up3, w_down3),
              (w_router4, w_gate_up4, w_down4)]

    y1 = y2 = w1p = w2p = None
    for li, (wr, wgu, wd) in enumerate(layers):
        if li == 0:
            d1s, d2s, d1g, d2g, w1p_n, w2p_n, meta = _route_first(x, wr)
            x_cur = x
        else:
            (d1s, d2s, d1g, d2g, w1p_n, w2p_n, meta,
             x_cur) = _route_combine(y1, y2, w1p, w2p, wr)
        buf = _get_dispatch()(d1s.reshape(T), d2s.reshape(T), x_cur)
        y = _ffn(meta.reshape(64), buf, wgu, wd)
        y1, y2 = _get_combine()(y, d1g.reshape(T), d2g.reshape(T))
        w1p, w2p = w1p_n, w2p_n

    out = _final_sum(y1, y2, w1p, w2p)
    return out.reshape(B, S, Dm)
